# Initial kernel scaffold; baseline (speedup 1.0000x reference)
import functools

import jax
import jax.numpy as jnp
from jax import lax
from jax.experimental import pallas as pl
from jax.experimental.pallas import tpu as pltpu
from jax.experimental.pallas import tpu_sc as plsc

_INFO = plsc.get_sparse_core_info()
_NC = _INFO.num_cores
_NS = _INFO.num_subcores
_NW = _NC * _NS

_CHUNK = 512


@jax.jit
def _gather_rows(table, idx_flat):
    B = idx_flat.shape[0]
    V, D = table.shape
    b_per_w = B // _NW
    n_chunks = b_per_w // _CHUNK
    mesh = plsc.VectorSubcoreMesh(core_axis_name="c", subcore_axis_name="s")

    @functools.partial(
        pl.kernel,
        mesh=mesh,
        out_type=jax.ShapeDtypeStruct((B, D), jnp.float32),
        scratch_types=[
            pltpu.VMEM((_CHUNK,), jnp.int32),
            pltpu.VMEM((_CHUNK, D), jnp.float32),
            pltpu.SemaphoreType.DMA,
        ],
    )
    def k(table_hbm, idx_hbm, out_hbm, idx_v, rows_v, sem):
        wid = lax.axis_index("s") * _NC + lax.axis_index("c")
        base = wid * b_per_w

        def body(g, carry):
            off = base + g * _CHUNK
            pltpu.sync_copy(idx_hbm.at[pl.ds(off, _CHUNK)], idx_v)
            pltpu.async_copy(table_hbm.at[idx_v], rows_v, sem).wait()
            pltpu.sync_copy(rows_v, out_hbm.at[pl.ds(off, _CHUNK)])
            return carry

        lax.fori_loop(0, n_chunks, body, 0)

    return k(table, idx_flat)


def kernel(indices, table):
    nb, nh = indices.shape
    idx_flat = indices.reshape(-1).astype(jnp.int32)
    out = _gather_rows(table, idx_flat)
    return out.reshape(nb, nh, table.shape[1])


# speedup vs baseline: 1.7974x; 1.7974x over previous
import functools

import jax
import jax.numpy as jnp
from jax import lax
from jax.experimental import pallas as pl
from jax.experimental.pallas import tpu as pltpu
from jax.experimental.pallas import tpu_sc as plsc

_INFO = plsc.get_sparse_core_info()
_NC = _INFO.num_cores
_NS = _INFO.num_subcores
_NW = _NC * _NS

_CHUNK = 512


@jax.jit
def _gather_rows(table, idx_flat):
    B = idx_flat.shape[0]
    V, D = table.shape
    b_per_w = B // _NW
    n_chunks = b_per_w // _CHUNK
    mesh = plsc.VectorSubcoreMesh(core_axis_name="c", subcore_axis_name="s")

    @functools.partial(
        pl.kernel,
        mesh=mesh,
        out_type=jax.ShapeDtypeStruct((B, D), jnp.float32),
        scratch_types=[
            pltpu.VMEM((_CHUNK,), jnp.int32),
            pltpu.VMEM((_CHUNK, D), jnp.float32),
            pltpu.SemaphoreType.DMA,
        ],
        compiler_params=pltpu.CompilerParams(use_tc_tiling_on_sc=False),
    )
    def k(table_hbm, idx_hbm, out_hbm, idx_v, rows_v, sem):
        wid = lax.axis_index("s") * _NC + lax.axis_index("c")
        base = wid * b_per_w

        def body(g, carry):
            off = base + g * _CHUNK
            pltpu.sync_copy(idx_hbm.at[pl.ds(off, _CHUNK)], idx_v)
            pltpu.async_copy(table_hbm.at[idx_v], rows_v, sem).wait()
            pltpu.sync_copy(rows_v, out_hbm.at[pl.ds(off, _CHUNK)])
            return carry

        lax.fori_loop(0, n_chunks, body, 0)

    return k(table, idx_flat)


def kernel(indices, table):
    nb, nh = indices.shape
    idx_flat = indices.reshape(-1).astype(jnp.int32)
    out = _gather_rows(table, idx_flat)
    return out.reshape(nb, nh, table.shape[1])


# speedup vs baseline: 1.8699x; 1.0403x over previous
import functools

import jax
import jax.numpy as jnp
from jax import lax
from jax.experimental import pallas as pl
from jax.experimental.pallas import tpu as pltpu
from jax.experimental.pallas import tpu_sc as plsc

_INFO = plsc.get_sparse_core_info()
_NC = _INFO.num_cores
_NS = _INFO.num_subcores
_NW = _NC * _NS

_CHUNK = 512
_NBUF = 2


@jax.jit
def _gather_rows(table, idx_flat):
    B = idx_flat.shape[0]
    V, D = table.shape
    b_per_w = B // _NW
    n_chunks = b_per_w // _CHUNK
    n_outer = n_chunks // _NBUF
    mesh = plsc.VectorSubcoreMesh(core_axis_name="c", subcore_axis_name="s")

    @functools.partial(
        pl.kernel,
        mesh=mesh,
        out_type=jax.ShapeDtypeStruct((B, D), jnp.float32),
        scratch_types=[
            pltpu.VMEM((_NBUF, _CHUNK), jnp.int32),
            pltpu.VMEM((_NBUF, _CHUNK, D), jnp.float32),
            [pltpu.SemaphoreType.DMA] * _NBUF,
            [pltpu.SemaphoreType.DMA] * _NBUF,
        ],
        compiler_params=pltpu.CompilerParams(use_tc_tiling_on_sc=False),
    )
    def k(table_hbm, idx_hbm, out_hbm, idx_v, rows_v, gsem, osem):
        wid = lax.axis_index("s") * _NC + lax.axis_index("c")
        base = wid * b_per_w

        def wait_gather(b):
            pltpu.make_async_copy(
                table_hbm.at[pl.ds(0, _CHUNK)], rows_v.at[b], gsem[b]
            ).wait()

        def wait_wb(b):
            pltpu.make_async_copy(
                table_hbm.at[pl.ds(0, _CHUNK)], rows_v.at[b], osem[b]
            ).wait()

        for b in range(_NBUF):
            off = base + b * _CHUNK
            pltpu.sync_copy(idx_hbm.at[pl.ds(off, _CHUNK)], idx_v.at[b])
            pltpu.async_copy(table_hbm.at[idx_v.at[b]], rows_v.at[b], gsem[b])

        def body(i, carry):
            for b in range(_NBUF):
                g = i * _NBUF + b
                off = base + g * _CHUNK
                wait_gather(b)
                pltpu.async_copy(rows_v.at[b], out_hbm.at[pl.ds(off, _CHUNK)], osem[b])
            for b in range(_NBUF):
                g = i * _NBUF + b

                @pl.when(g + _NBUF < n_chunks)
                def _():
                    off2 = base + (g + _NBUF) * _CHUNK
                    wait_wb(b)
                    pltpu.sync_copy(idx_hbm.at[pl.ds(off2, _CHUNK)], idx_v.at[b])
                    pltpu.async_copy(table_hbm.at[idx_v.at[b]], rows_v.at[b], gsem[b])

            return carry

        lax.fori_loop(0, n_outer, body, 0)

        for b in range(_NBUF):
            wait_wb(b)

    return k(table, idx_flat)


def kernel(indices, table):
    nb, nh = indices.shape
    idx_flat = indices.reshape(-1).astype(jnp.int32)
    out = _gather_rows(table, idx_flat)
    return out.reshape(nb, nh, table.shape[1])


# speedup vs baseline: 1.8879x; 1.0096x over previous
import functools

import jax
import jax.numpy as jnp
from jax import lax
from jax.experimental import pallas as pl
from jax.experimental.pallas import tpu as pltpu
from jax.experimental.pallas import tpu_sc as plsc

_INFO = plsc.get_sparse_core_info()
_NC = _INFO.num_cores
_NS = _INFO.num_subcores
_NW = _NC * _NS

_CHUNK = 320
_NBUF = 4


@jax.jit
def _gather_rows(table, idx_flat):
    B = idx_flat.shape[0]
    V, D = table.shape
    b_per_w = B // _NW
    n_chunks = b_per_w // _CHUNK
    n_outer = n_chunks // _NBUF
    mesh = plsc.VectorSubcoreMesh(core_axis_name="c", subcore_axis_name="s")

    @functools.partial(
        pl.kernel,
        mesh=mesh,
        out_type=jax.ShapeDtypeStruct((B, D), jnp.float32),
        scratch_types=[
            pltpu.VMEM((_NBUF, _CHUNK), jnp.int32),
            pltpu.VMEM((_NBUF, _CHUNK, D), jnp.float32),
            [pltpu.SemaphoreType.DMA] * _NBUF,
            [pltpu.SemaphoreType.DMA] * _NBUF,
        ],
        compiler_params=pltpu.CompilerParams(use_tc_tiling_on_sc=False),
    )
    def k(table_hbm, idx_hbm, out_hbm, idx_v, rows_v, gsem, osem):
        wid = lax.axis_index("s") * _NC + lax.axis_index("c")
        base = wid * b_per_w

        def wait_gather(b):
            pltpu.make_async_copy(
                table_hbm.at[pl.ds(0, _CHUNK)], rows_v.at[b], gsem[b]
            ).wait()

        def wait_wb(b):
            pltpu.make_async_copy(
                table_hbm.at[pl.ds(0, _CHUNK)], rows_v.at[b], osem[b]
            ).wait()

        for b in range(_NBUF):
            off = base + b * _CHUNK
            pltpu.sync_copy(idx_hbm.at[pl.ds(off, _CHUNK)], idx_v.at[b])
            pltpu.async_copy(table_hbm.at[idx_v.at[b]], rows_v.at[b], gsem[b])

        def body(i, carry):
            for b in range(_NBUF):
                g = i * _NBUF + b
                off = base + g * _CHUNK
                wait_gather(b)
                pltpu.async_copy(rows_v.at[b], out_hbm.at[pl.ds(off, _CHUNK)], osem[b])
            for b in range(_NBUF):
                g = i * _NBUF + b

                @pl.when(g + _NBUF < n_chunks)
                def _():
                    off2 = base + (g + _NBUF) * _CHUNK
                    wait_wb(b)
                    pltpu.sync_copy(idx_hbm.at[pl.ds(off2, _CHUNK)], idx_v.at[b])
                    pltpu.async_copy(table_hbm.at[idx_v.at[b]], rows_v.at[b], gsem[b])

            return carry

        lax.fori_loop(0, n_outer, body, 0)

        for b in range(_NBUF):
            wait_wb(b)

    return k(table, idx_flat)


def kernel(indices, table):
    nb, nh = indices.shape
    idx_flat = indices.reshape(-1).astype(jnp.int32)
    out = _gather_rows(table, idx_flat)
    return out.reshape(nb, nh, table.shape[1])
